# Initial kernel scaffold; baseline (speedup 1.0000x reference)
import functools

import jax
import jax.numpy as jnp
from jax import lax
from jax.experimental import pallas as pl
from jax.experimental.pallas import tpu as pltpu
from jax.experimental.pallas import tpu_sc as plsc

D = 64
MEM = 4194304
KNUTH = 2654435761

NC = 2
NS = 16
NW = NC * NS
L = 16


def _i32(v: int) -> jnp.int32:
    v &= 0xFFFFFFFF
    return jnp.int32(v - (1 << 32) if v >= (1 << 31) else v)


K_I32 = _i32(KNUTH)
C64K_I32 = _i32(64 * KNUTH)
MASK_I32 = _i32(MEM - 1)


def _sc_gather(xf, mem_pool, *, cx):
    nx = xf.shape[0]
    assert nx % (NW * cx) == 0
    nx_w = nx // NW
    nchunk = nx_w // cx
    co = cx * D

    mesh = plsc.VectorSubcoreMesh(core_axis_name="c", subcore_axis_name="s")

    @functools.partial(
        pl.kernel,
        out_type=jax.ShapeDtypeStruct((nx * D,), jnp.float32),
        mesh=mesh,
        scratch_types=[
            pltpu.VMEM((cx,), jnp.int32),
            pltpu.VMEM((co,), jnp.int32),
            pltpu.VMEM((co,), jnp.float32),
            pltpu.VMEM((4 * L,), jnp.int32),
            pltpu.SemaphoreType.DMA,
        ],
    )
    def k(x_hbm, pool_hbm, out_hbm, x_v, idx_v, dat_v, ct_v, sem):
        wid = lax.axis_index("s") * NC + lax.axis_index("c")
        id_base = wid * nx_w

        lanes = lax.iota(jnp.int32, L)
        for t in range(4):
            ct_v[pl.ds(t * L, L)] = (lanes + t * L) * K_I32

        def chunk_body(ci):
            idb = id_base + ci * cx
            pltpu.sync_copy(x_hbm.at[pl.ds(idb, cx)], x_v)

            def id_body(j, _):
                s = x_v[j] * C64K_I32
                sb = jnp.full((L,), s, jnp.int32)
                for t in range(4):
                    h = sb + ct_v[pl.ds(t * L, L)]
                    h = h ^ lax.shift_right_logical(h, 16)
                    idx_v[pl.ds(j * D + t * L, L)] = h & MASK_I32
                return 0

            lax.fori_loop(0, cx, id_body, 0)
            pltpu.async_copy(pool_hbm.at[idx_v], dat_v, sem).wait()
            pltpu.sync_copy(dat_v, out_hbm.at[pl.ds(idb * D, co)])

        pl.loop(0, nchunk)(chunk_body)

    return k(xf, mem_pool)


def kernel(x, mem_pool):
    b, f = x.shape
    xf = x.reshape(b * f)
    out = _sc_gather(xf, mem_pool, cx=512)
    return out.reshape(b, f, D)


# speedup vs baseline: 334.3701x; 334.3701x over previous
import functools

import jax
import jax.numpy as jnp
import numpy as np
from jax import lax
from jax.experimental import pallas as pl
from jax.experimental.pallas import tpu as pltpu
from jax.experimental.pallas import tpu_sc as plsc

D = 64
MEM = 4194304
KNUTH = 2654435761

NC = 2
NS = 16
NW = NC * NS
L = 16


def _i32(v: int) -> np.int32:
    v &= 0xFFFFFFFF
    return np.int32(v - (1 << 32) if v >= (1 << 31) else v)


K_I32 = _i32(KNUTH)
C64K_I32 = _i32(64 * KNUTH)
MASK_I32 = _i32(MEM - 1)


def _sc_gather(xf, mem_pool, *, cx):
    nx = xf.shape[0]
    assert nx % (NW * cx) == 0
    nx_w = nx // NW
    nchunk = nx_w // cx
    co = cx * D

    mesh = plsc.VectorSubcoreMesh(core_axis_name="c", subcore_axis_name="s")

    @functools.partial(
        pl.kernel,
        out_type=jax.ShapeDtypeStruct((nx * D,), jnp.float32),
        mesh=mesh,
        scratch_types=[
            pltpu.VMEM((cx,), jnp.int32),
            pltpu.VMEM((co,), jnp.int32),
            pltpu.VMEM((co,), jnp.float32),
            pltpu.SemaphoreType.DMA,
        ],
    )
    def k(x_hbm, pool_hbm, out_hbm, x_v, idx_v, dat_v, sem):
        wid = lax.axis_index("s") * NC + lax.axis_index("c")
        id_base = wid * nx_w

        lanes = lax.iota(jnp.int32, L)
        cts = [(lanes + t * L) * K_I32 for t in range(4)]

        def chunk_body(ci):
            idb = id_base + ci * cx
            pltpu.sync_copy(x_hbm.at[pl.ds(idb, cx)], x_v)

            def grp_body(g, _):
                sv = x_v[pl.ds(g * L, L)] * C64K_I32
                for lane in range(L):
                    sb = jnp.full((L,), sv[lane], jnp.int32)
                    for t in range(4):
                        h = sb + cts[t]
                        h = h ^ lax.shift_right_logical(h, 16)
                        idx_v[pl.ds((g * L + lane) * D + t * L, L)] = h & MASK_I32
                return 0

            lax.fori_loop(0, cx // L, grp_body, 0)
            pltpu.async_copy(pool_hbm.at[idx_v], dat_v, sem).wait()
            pltpu.sync_copy(dat_v, out_hbm.at[pl.ds(idb * D, co)])

        pl.loop(0, nchunk)(chunk_body)

    return k(xf, mem_pool)


def kernel(x, mem_pool):
    b, f = x.shape
    xf = x.reshape(b * f)
    out = _sc_gather(xf, mem_pool, cx=512)
    return out.reshape(b, f, D)


# speedup vs baseline: 356.8380x; 1.0672x over previous
import functools

import jax
import jax.numpy as jnp
import numpy as np
from jax import lax
from jax.experimental import pallas as pl
from jax.experimental.pallas import tpu as pltpu
from jax.experimental.pallas import tpu_sc as plsc

D = 64
MEM = 4194304
KNUTH = 2654435761

NC = 2
NS = 16
NW = NC * NS
L = 16


def _i32(v: int) -> np.int32:
    v &= 0xFFFFFFFF
    return np.int32(v - (1 << 32) if v >= (1 << 31) else v)


K_I32 = _i32(KNUTH)
C64K_I32 = _i32(64 * KNUTH)
MASK_I32 = _i32(MEM - 1)


def _sc_gather(xf, mem_pool, *, cx):
    nx = xf.shape[0]
    assert nx % (NW * cx) == 0
    nx_w = nx // NW
    nchunk = nx_w // cx
    assert nchunk % 2 == 0 and nchunk >= 4
    co = cx * D

    mesh = plsc.VectorSubcoreMesh(core_axis_name="c", subcore_axis_name="s")

    @functools.partial(
        pl.kernel,
        out_type=jax.ShapeDtypeStruct((nx * D,), jnp.float32),
        mesh=mesh,
        scratch_types=[
            pltpu.VMEM((cx,), jnp.int32),
            pltpu.VMEM((co,), jnp.int32),
            pltpu.VMEM((co,), jnp.int32),
            pltpu.VMEM((co,), jnp.float32),
            pltpu.VMEM((co,), jnp.float32),
            pltpu.SemaphoreType.DMA,
            pltpu.SemaphoreType.DMA,
            pltpu.SemaphoreType.DMA,
            pltpu.SemaphoreType.DMA,
        ],
    )
    def k(x_hbm, pool_hbm, out_hbm, x_v, i0, i1, d0, d1, gs0, gs1, ws0, ws1):
        wid = lax.axis_index("s") * NC + lax.axis_index("c")
        id_base = wid * nx_w

        lanes = lax.iota(jnp.int32, L)
        cts = [(lanes + t * L) * K_I32 for t in range(4)]

        def compute(c, ib):
            pltpu.sync_copy(x_hbm.at[pl.ds(id_base + c * cx, cx)], x_v)

            def grp_body(g, _):
                sv = x_v[pl.ds(g * L, L)] * C64K_I32
                for lane in range(L):
                    sb = jnp.full((L,), sv[lane], jnp.int32)
                    for t in range(4):
                        h = sb + cts[t]
                        h = h ^ lax.shift_right_logical(h, 16)
                        ib[pl.ds((g * L + lane) * D + t * L, L)] = h & MASK_I32
                return 0

            lax.fori_loop(0, cx // L, grp_body, 0)

        def fire_g(ib, db, gs):
            pltpu.async_copy(pool_hbm.at[ib], db, gs)

        def wait_g(ib, db, gs):
            pltpu.make_async_copy(pool_hbm.at[ib], db, gs).wait()

        def fire_w(c, db, ws):
            pltpu.async_copy(db, out_hbm.at[pl.ds((id_base + c * cx) * D, co)], ws)

        def wait_w(db, ws):
            pltpu.make_async_copy(db, out_hbm.at[pl.ds(id_base * D, co)], ws).wait()

        compute(0, i0)
        fire_g(i0, d0, gs0)
        compute(1, i1)
        fire_g(i1, d1, gs1)

        def pair_body(p):
            c = 2 * p
            wait_g(i0, d0, gs0)
            fire_w(c, d0, ws0)
            compute(c + 2, i0)
            wait_w(d0, ws0)
            fire_g(i0, d0, gs0)
            wait_g(i1, d1, gs1)
            fire_w(c + 1, d1, ws1)
            compute(c + 3, i1)
            wait_w(d1, ws1)
            fire_g(i1, d1, gs1)

        pl.loop(0, (nchunk - 2) // 2)(pair_body)

        ce = nchunk - 2
        wait_g(i0, d0, gs0)
        fire_w(ce, d0, ws0)
        wait_g(i1, d1, gs1)
        fire_w(ce + 1, d1, ws1)
        wait_w(d0, ws0)
        wait_w(d1, ws1)

    return k(xf, mem_pool)


def kernel(x, mem_pool):
    b, f = x.shape
    xf = x.reshape(b * f)
    out = _sc_gather(xf, mem_pool, cx=416)
    return out.reshape(b, f, D)
